# Initial kernel scaffold; baseline (speedup 1.0000x reference)
import jax
import jax.numpy as jnp
from jax.experimental import pallas as pl


def kernel(x, conv1_w, conv1_b, conv2_w, conv2_b, res_w1, res_w2, preq_w, preq_b, codebook):
    raise NotImplementedError("write your pallas kernel here")


# speedup vs baseline: 1.1090x; 1.1090x over previous
import jax
import jax.numpy as jnp
from jax.experimental import pallas as pl
from jax.experimental.pallas import tpu as pltpu

_NE = 1024
_ED = 64
_BETA = 0.25
_TOK = 1024


def _k1_body(a_ref, w_ref, b_ref, o_ref):
    o_ref[...] = jnp.maximum(
        jnp.dot(a_ref[...], w_ref[...], preferred_element_type=jnp.float32)
        + b_ref[...], 0.0)


def _shift_tokens(hr, dy, dx):
    s = 32 * dy + dx
    if s > 0:
        sh = jnp.concatenate(
            [hr[s:], jnp.zeros((s, hr.shape[1]), jnp.float32)], axis=0)
    elif s < 0:
        sh = jnp.concatenate(
            [jnp.zeros((-s, hr.shape[1]), jnp.float32), hr[:1024 + s]], axis=0)
    else:
        sh = hr
    if dx != 0:
        xo = jax.lax.broadcasted_iota(jnp.int32, (1024, 1), 0) % 32
        valid = (xo + dx >= 0) & (xo + dx < 32)
        sh = jnp.where(valid, sh, 0.0)
    return sh


def _k2_body(a_ref, w2_ref, b2_ref, wr1_ref, wr2_ref, wp_ref, bp_ref,
             c_ref, ct_ref, csq_ref,
             zq_ref, idx_ref, loss_ref, perp_ref,
             cnt_ref, sse_ref):
    n = pl.program_id(0)

    h = jnp.dot(a_ref[...], w2_ref[...],
                preferred_element_type=jnp.float32) + b2_ref[...]

    for _ in range(2):
        hr = jnp.maximum(h, 0.0)
        acc = jnp.zeros((1024, 64), jnp.float32)
        k = 0
        for ky in range(3):
            for kx in range(3):
                sh = _shift_tokens(hr, ky - 1, kx - 1)
                acc = acc + jnp.dot(sh, wr1_ref[k],
                                    preferred_element_type=jnp.float32)
                k += 1
        r = jnp.dot(jnp.maximum(acc, 0.0), wr2_ref[...],
                    preferred_element_type=jnp.float32)
        h = h + r

    h = jnp.maximum(h, 0.0)
    zf = jnp.dot(h, wp_ref[...],
                 preferred_element_type=jnp.float32) + bp_ref[...]

    scores = csq_ref[...] - 2.0 * jnp.dot(
        zf, ct_ref[...], preferred_element_type=jnp.float32)
    m = jnp.min(scores, axis=1, keepdims=True)
    jj = jax.lax.broadcasted_iota(jnp.int32, (1024, _NE), 1)
    idx = jnp.min(jnp.where(scores <= m, jj, _NE), axis=1, keepdims=True)

    onehot = (jj == idx).astype(jnp.float32)
    zq1 = jnp.dot(onehot, c_ref[...], preferred_element_type=jnp.float32)

    idx_ref[...] = idx[None]
    zq_ref[...] = jnp.transpose(zq1)[None]

    @pl.when(n == 0)
    def _():
        cnt_ref[...] = jnp.zeros_like(cnt_ref)
        sse_ref[0, 0] = 0.0

    cnt_ref[...] += jnp.sum(onehot, axis=0, keepdims=True)
    sse_ref[0, 0] += jnp.sum((zq1 - zf) ** 2)

    @pl.when(n == pl.num_programs(0) - 1)
    def _():
        total = sse_ref[0, 0]
        loss_ref[...] = jnp.full(
            (1, 1), (1.0 + _BETA) * total / (8.0 * _TOK * _ED), jnp.float32)
        e_mean = cnt_ref[...] / (8.0 * _TOK)
        ent = jnp.sum(e_mean * jnp.log(e_mean + 1e-10))
        perp_ref[...] = jnp.full((1, 1), jnp.exp(-ent), jnp.float32)


def kernel(x, conv1_w, conv1_b, conv2_w, conv2_b, res_w1, res_w2,
           preq_w, preq_b, codebook):
    f32 = jnp.float32

    a1 = x.reshape(8, 3, 128, 4, 128, 4).transpose(0, 2, 4, 1, 3, 5)
    a1 = a1.reshape(8 * 128 * 128, 48)
    w1 = conv1_w.reshape(64, 48).T
    b1 = conv1_b.reshape(1, 64)

    h1 = pl.pallas_call(
        _k1_body,
        grid=(16,),
        in_specs=[
            pl.BlockSpec((8192, 48), lambda i: (i, 0)),
            pl.BlockSpec((48, 64), lambda i: (0, 0)),
            pl.BlockSpec((1, 64), lambda i: (0, 0)),
        ],
        out_specs=pl.BlockSpec((8192, 64), lambda i: (i, 0)),
        out_shape=jax.ShapeDtypeStruct((131072, 64), f32),
    )(a1, w1, b1)

    a2 = h1.reshape(8, 32, 4, 32, 4, 64).transpose(0, 1, 3, 5, 2, 4)
    a2 = a2.reshape(8 * 1024, 1024)

    w2 = conv2_w.reshape(128, 1024).T
    b2 = conv2_b.reshape(1, 128)
    wr1 = res_w1.transpose(2, 3, 1, 0).reshape(9, 128, 64)
    wr2 = res_w2.reshape(128, 64).T
    wp = preq_w.reshape(64, 128).T
    bp = preq_b.reshape(1, 64)
    ct = codebook.T
    csq = jnp.sum(codebook ** 2, axis=1).reshape(1, _NE)

    zq, idx, loss, perp = pl.pallas_call(
        _k2_body,
        grid=(8,),
        in_specs=[
            pl.BlockSpec((_TOK, 1024), lambda n: (n, 0)),
            pl.BlockSpec((1024, 128), lambda n: (0, 0)),
            pl.BlockSpec((1, 128), lambda n: (0, 0)),
            pl.BlockSpec((9, 128, 64), lambda n: (0, 0, 0)),
            pl.BlockSpec((64, 128), lambda n: (0, 0)),
            pl.BlockSpec((128, 64), lambda n: (0, 0)),
            pl.BlockSpec((1, 64), lambda n: (0, 0)),
            pl.BlockSpec((_NE, _ED), lambda n: (0, 0)),
            pl.BlockSpec((_ED, _NE), lambda n: (0, 0)),
            pl.BlockSpec((1, _NE), lambda n: (0, 0)),
        ],
        out_specs=[
            pl.BlockSpec((1, _ED, _TOK), lambda n: (n, 0, 0)),
            pl.BlockSpec((1, _TOK, 1), lambda n: (n, 0, 0)),
            pl.BlockSpec((1, 1), lambda n: (0, 0)),
            pl.BlockSpec((1, 1), lambda n: (0, 0)),
        ],
        out_shape=[
            jax.ShapeDtypeStruct((8, _ED, _TOK), f32),
            jax.ShapeDtypeStruct((8, _TOK, 1), jnp.int32),
            jax.ShapeDtypeStruct((1, 1), f32),
            jax.ShapeDtypeStruct((1, 1), f32),
        ],
        scratch_shapes=[
            pltpu.VMEM((1, _NE), f32),
            pltpu.SMEM((1, 1), f32),
        ],
    )(a2, w2, b2, wr1, wr2, wp, bp, codebook, ct, csq)

    z_q = zq.reshape(8, _ED, 32, 32)
    idx_out = idx.reshape(8 * _TOK, 1)
    return (loss[0, 0], z_q, perp[0, 0], codebook, idx_out)
